# baseline (speedup 1.0000x reference)
import functools

import jax
import jax.numpy as jnp
from jax import lax
from jax.experimental import pallas as pl
from jax.experimental.pallas import tpu as pltpu
from jax.experimental.pallas import tpu_sc as plsc

VOCAB = 1000
DIM = 16
BATCH = 16384

_info = plsc.get_sparse_core_info()
_NC, _NS, _L = _info.num_cores, _info.num_subcores, _info.num_lanes
_NW = _NC * _NS
_BPW = BATCH // _NW
_GROUPS = _BPW // _L

_mesh = plsc.VectorSubcoreMesh(core_axis_name="c", subcore_axis_name="s")


@functools.partial(
    pl.kernel,
    mesh=_mesh,
    compiler_params=pltpu.CompilerParams(needs_layout_passes=False),
    out_type=jax.ShapeDtypeStruct((BATCH, DIM), jnp.float32),
    scratch_types=[
        pltpu.VMEM((128, 128), jnp.float32),
        pltpu.VMEM((_BPW,), jnp.int32),
        pltpu.VMEM((_BPW, DIM), jnp.float32),
    ],
)
def _lookup_gather(idx_hbm, table_hbm, out_hbm, table_v, idx_v, rows_v):
    wid = lax.axis_index("s") * _NC + lax.axis_index("c")
    base = wid * _BPW
    pltpu.sync_copy(table_hbm, table_v)
    pltpu.sync_copy(idx_hbm.at[pl.ds(base, _BPW)], idx_v)
    lane = lax.iota(jnp.int32, 16)
    for g in range(_GROUPS):
        v = idx_v[pl.ds(g * _L, _L)]
        ok = (v >= 0) & (v < VOCAB)
        m = jnp.where(ok, v + 1, 0)
        r = m >> 3
        c0 = (m & 7) << 4
        row_ids = lane + (g * _L)
        for d in range(DIM):
            col = plsc.load_gather(table_v, [r, c0 + d])
            plsc.store_scatter(
                rows_v, [row_ids, jnp.full((16,), d, jnp.int32)], col
            )
    pltpu.sync_copy(rows_v, out_hbm.at[pl.ds(base, _BPW)])


def kernel(inputs, table):
    idx = inputs.reshape(BATCH)
    packed = jnp.pad(table, ((0, 1024 - (VOCAB + 1)), (0, 0))).reshape(128, 128)
    return _lookup_gather(idx, packed)


# speedup vs baseline: 1.1104x; 1.1104x over previous
import functools

import jax
import jax.numpy as jnp
from jax import lax
from jax.experimental import pallas as pl
from jax.experimental.pallas import tpu as pltpu
from jax.experimental.pallas import tpu_sc as plsc

VOCAB = 1000
DIM = 16
BATCH = 16384

_info = plsc.get_sparse_core_info()
_NC, _NS, _L = _info.num_cores, _info.num_subcores, _info.num_lanes
_NW = _NC * _NS
_BPW = BATCH // _NW
_GROUPS = _BPW // _L

_mesh = plsc.VectorSubcoreMesh(core_axis_name="c", subcore_axis_name="s")

_DNUMS = lax.GatherDimensionNumbers(
    offset_dims=(), collapsed_slice_dims=(0,), start_index_map=(0,)
)


def _bcast_lane(vec, k):
    idx = jnp.full((_L, 1), k, jnp.int32)
    return lax.gather(
        vec, idx, _DNUMS, slice_sizes=(1,),
        mode=lax.GatherScatterMode.PROMISE_IN_BOUNDS,
    )


@functools.partial(
    pl.kernel,
    mesh=_mesh,
    compiler_params=pltpu.CompilerParams(needs_layout_passes=False),
    out_type=jax.ShapeDtypeStruct((BATCH, DIM), jnp.float32),
    scratch_types=[
        pltpu.VMEM((128, 128), jnp.float32),
        pltpu.VMEM((_BPW,), jnp.int32),
        pltpu.VMEM((_BPW, DIM), jnp.float32),
    ],
)
def _lookup_gather(idx_hbm, table_hbm, out_hbm, table_v, idx_v, rows_v):
    wid = lax.axis_index("s") * _NC + lax.axis_index("c")
    base = wid * _BPW
    pltpu.sync_copy(table_hbm, table_v)
    pltpu.sync_copy(idx_hbm.at[pl.ds(base, _BPW)], idx_v)
    lane = lax.iota(jnp.int32, _L)
    for g in range(_GROUPS):
        v = idx_v[pl.ds(g * _L, _L)]
        ok = (v >= 0) & (v < VOCAB)
        m = jnp.where(ok, v + 1, 0)
        for k in range(_L):
            mb = _bcast_lane(m, k)
            row = plsc.load_gather(
                table_v, [mb >> 3, ((mb & 7) << 4) + lane]
            )
            rows_v[g * _L + k, :] = row
    pltpu.sync_copy(rows_v, out_hbm.at[pl.ds(base, _BPW)])


def kernel(inputs, table):
    idx = inputs.reshape(BATCH)
    packed = jnp.pad(table, ((0, 1024 - (VOCAB + 1)), (0, 0))).reshape(128, 128)
    return _lookup_gather(idx, packed)


# speedup vs baseline: 1.2945x; 1.1657x over previous
import functools

import jax
import jax.numpy as jnp
from jax import lax
from jax.experimental import pallas as pl
from jax.experimental.pallas import tpu as pltpu
from jax.experimental.pallas import tpu_sc as plsc

VOCAB = 1000
DIM = 16
BATCH = 16384

_info = plsc.get_sparse_core_info()
_NC, _NS, _L = _info.num_cores, _info.num_subcores, _info.num_lanes
_NW = _NC * _NS
_BPW = BATCH // _NW
_GROUPS = _BPW // _L

_mesh = plsc.VectorSubcoreMesh(core_axis_name="c", subcore_axis_name="s")

_DNUMS = lax.GatherDimensionNumbers(
    offset_dims=(), collapsed_slice_dims=(0,), start_index_map=(0,)
)


def _bcast_lane(vec, k):
    idx = jnp.full((_L,), k, jnp.int32).reshape(_L, 1)
    return lax.gather(
        vec, idx, _DNUMS, slice_sizes=(1,),
        mode=lax.GatherScatterMode.PROMISE_IN_BOUNDS,
    )


@functools.partial(
    pl.kernel,
    mesh=_mesh,
    compiler_params=pltpu.CompilerParams(needs_layout_passes=False),
    out_type=jax.ShapeDtypeStruct((BATCH, DIM), jnp.float32),
    scratch_types=[
        pltpu.VMEM((128, 128), jnp.float32),
        pltpu.VMEM((_BPW,), jnp.int32),
        pltpu.VMEM((_BPW, DIM), jnp.float32),
    ],
)
def _lookup_gather(idx_hbm, table_hbm, out_hbm, table_v, idx_v, rows_v):
    wid = lax.axis_index("s") * _NC + lax.axis_index("c")
    base = wid * _BPW
    pltpu.sync_copy(table_hbm, table_v)
    pltpu.sync_copy(idx_hbm.at[pl.ds(base, _BPW)], idx_v)
    lane = lax.iota(jnp.int32, _L)
    for g in range(_GROUPS):
        v = idx_v[pl.ds(g * _L, _L)]
        ok = (v >= 0) & (v < VOCAB)
        idx_v[pl.ds(g * _L, _L)] = jnp.where(ok, v + 1, 0)

    @plsc.parallel_loop(0, _BPW, 1, unroll=8)
    def row_body(i):
        m = idx_v[pl.ds((i >> 4) << 4, _L)]
        mb = _bcast_lane(m, i & 15)
        row = plsc.load_gather(table_v, [mb >> 3, ((mb & 7) << 4) + lane])
        rows_v[i, :] = row
    pltpu.sync_copy(rows_v, out_hbm.at[pl.ds(base, _BPW)])


def kernel(inputs, table):
    idx = inputs.reshape(BATCH)
    packed = jnp.pad(table, ((0, 1024 - (VOCAB + 1)), (0, 0))).reshape(128, 128)
    return _lookup_gather(idx, packed)
